# baseline (speedup 1.0000x reference)
import functools

import jax
import jax.numpy as jnp
from jax import lax
from jax.experimental import pallas as pl
from jax.experimental.pallas import tpu as pltpu
from jax.experimental.pallas import tpu_sc as plsc

D = 256
N_NODES = 100000
N_REG = 10000
B = 50000

NC = 2
NS = 16
NW = NC * NS

CHUNK = 112
N_CHUNKS = 14
B_W = CHUNK * N_CHUNKS
B_PAD = B_W * NW
PAD = B_PAD - B

REG_W = 312
REG_CHUNK = 104
REG_NCHUNKS = 3
REG_TAIL = N_REG - REG_W * NW

NV = D // 16


UNROLL = 4


def _accum(buf_ref, nrows, acc):
    def body(k, a):
        r = k * UNROLL
        for dr in range(UNROLL):
            a = tuple(a[i] + buf_ref[r + dr, pl.ds(i * 16, 16)]
                      for i in range(NV))
        return a
    return lax.fori_loop(0, nrows // UNROLL, body, acc)


@functools.partial(
    pl.kernel,
    mesh=plsc.VectorSubcoreMesh(core_axis_name="c", subcore_axis_name="s"),
    out_type=jax.ShapeDtypeStruct((NW, 2 * D), jnp.float32),
    scratch_types=[
        pltpu.VMEM((N_CHUNKS, CHUNK), jnp.int32),
        pltpu.VMEM((CHUNK, D), jnp.float32),
        pltpu.VMEM((CHUNK, D), jnp.float32),
        pltpu.VMEM((CHUNK, D), jnp.float32),
        pltpu.VMEM((1, D), jnp.float32),
        pltpu.VMEM((2 * D,), jnp.float32),
        pltpu.SemaphoreType.DMA,
        pltpu.SemaphoreType.DMA,
        pltpu.SemaphoreType.DMA,
    ],
)
def _sc_sums(node_hbm, region_hbm, idx_hbm, out_hbm,
             idx_v, buf0, buf1, buf2, row_v, out_v, sem0, sem1, sem2):
    wid = lax.axis_index("s") * NC + lax.axis_index("c")
    bufs = (buf0, buf1)
    sems = (sem0, sem1)
    zeros = tuple(jnp.zeros((16,), jnp.float32) for _ in range(NV))

    base = wid * REG_W
    reg_prefetch = pltpu.async_copy(
        region_hbm.at[pl.ds(base, REG_CHUNK)], buf2.at[pl.ds(0, REG_CHUNK)], sem2)

    pltpu.sync_copy(idx_hbm.at[wid], idx_v)
    handles = [None, None]
    handles[0] = pltpu.async_copy(node_hbm.at[idx_v.at[0]], buf0, sem0)
    acc_b = zeros
    for c in range(N_CHUNKS):
        if c + 1 < N_CHUNKS:
            nb = (c + 1) % 2
            handles[nb] = pltpu.async_copy(
                node_hbm.at[idx_v.at[c + 1]], bufs[nb], sems[nb])
        handles[c % 2].wait()
        acc_b = _accum(bufs[c % 2], CHUNK, acc_b)

    pltpu.sync_copy(node_hbm.at[pl.ds(0, 1)], row_v)
    w = jnp.float32(PAD / NW)
    acc_b = tuple(acc_b[i] - w * row_v[0, pl.ds(i * 16, 16)] for i in range(NV))

    rbufs = (buf2, buf0, buf1)
    rsems = (sem2, sem0, sem1)
    rhandles = [reg_prefetch, None, None]
    acc_r = zeros
    for c in range(REG_NCHUNKS):
        if c + 1 < REG_NCHUNKS:
            rhandles[c + 1] = pltpu.async_copy(
                region_hbm.at[pl.ds(base + (c + 1) * REG_CHUNK, REG_CHUNK)],
                rbufs[c + 1].at[pl.ds(0, REG_CHUNK)], rsems[c + 1])
        rhandles[c].wait()
        acc_r = _accum(rbufs[c], REG_CHUNK, acc_r)

    pltpu.sync_copy(
        region_hbm.at[pl.ds(REG_W * NW + lax.rem(wid, REG_TAIL), 1)], row_v)
    half = jnp.float32(0.5)
    acc_r = tuple(acc_r[i] + half * row_v[0, pl.ds(i * 16, 16)] for i in range(NV))

    for i in range(NV):
        out_v[pl.ds(i * 16, 16)] = acc_r[i]
        out_v[pl.ds(D + i * 16, 16)] = acc_b[i]
    pltpu.sync_copy(out_v, out_hbm.at[wid])


def _head_body(partials, W1, b1, W2, b2, W3, b3, W4, b4, W5, b5, W6, b6, out):
    s = jnp.sum(partials[...], axis=0)
    gs = (s[:D] * jnp.float32(1.0 / N_REG)).reshape(1, D)
    bm = (s[D:] * jnp.float32(1.0 / B)).reshape(1, D)
    hi = lax.Precision.HIGHEST
    f32 = jnp.float32
    gs = jnp.maximum(jnp.dot(gs, W1[...], precision=hi, preferred_element_type=f32) + b1[...], 0.0)
    gs = jnp.maximum(jnp.dot(gs, W2[...], precision=hi, preferred_element_type=f32) + b2[...], 0.0)
    bi = jnp.maximum(jnp.dot(bm, W3[...], precision=hi, preferred_element_type=f32) + b3[...], 0.0)
    bi = jnp.dot(bi, W4[...], precision=hi, preferred_element_type=f32) + b4[...]
    w5 = W5[...]
    h = jnp.dot(gs, w5[:128], precision=hi, preferred_element_type=f32)
    h = h + jnp.dot(bi, w5[128:], precision=hi, preferred_element_type=f32)
    h = jnp.maximum(h + b5[...], 0.0)
    v = jnp.dot(h, W6[...], precision=hi, preferred_element_type=f32) + b6[...]
    out[...] = v


def kernel(node_embeddings, region_embeddings, boundary_nodes,
           W1, b1, W2, b2, W3, b3, W4, b4, W5, b5, W6, b6):
    idx = boundary_nodes.astype(jnp.int32)
    idx_pad = jnp.concatenate([idx, jnp.zeros((PAD,), jnp.int32)])
    idx3d = idx_pad.reshape(NW, N_CHUNKS, CHUNK)

    partials = _sc_sums(node_embeddings, region_embeddings, idx3d)

    out = pl.pallas_call(
        _head_body,
        out_shape=jax.ShapeDtypeStruct((1, 1), jnp.float32),
    )(partials, W1, b1, W2, b2, W3, b3, W4, b4, W5, b5, W6, b6)
    return out.reshape(1)


# speedup vs baseline: 1.1027x; 1.1027x over previous
import functools

import jax
import jax.numpy as jnp
from jax import lax
from jax.experimental import pallas as pl
from jax.experimental.pallas import tpu as pltpu
from jax.experimental.pallas import tpu_sc as plsc

D = 256
N_NODES = 100000
N_REG = 10000
B = 50000

NC = 2
NS = 16
NW = NC * NS

CHUNK = 112
N_CHUNKS = 14
B_W = CHUNK * N_CHUNKS
B_LAST = B - (NW - 1) * B_W
PAD = NW * B_W - B

REG_W = 312
REG_CHUNK = 104
REG_NCHUNKS = 3
REG_TAIL = N_REG - REG_W * NW

NV = D // 16


def _accum(buf_ref, nrows, acc):
    def body(r, a):
        return tuple(a[i] + buf_ref[r, pl.ds(i * 16, 16)] for i in range(NV))
    return lax.fori_loop(0, nrows, body, acc)


@functools.partial(
    pl.kernel,
    mesh=plsc.VectorSubcoreMesh(core_axis_name="c", subcore_axis_name="s"),
    out_type=jax.ShapeDtypeStruct((NW, 2 * D), jnp.float32),
    scratch_types=[
        pltpu.VMEM((B_W,), jnp.int32),
        pltpu.VMEM((CHUNK, D), jnp.float32),
        pltpu.VMEM((CHUNK, D), jnp.float32),
        pltpu.VMEM((CHUNK, D), jnp.float32),
        pltpu.VMEM((1, D), jnp.float32),
        pltpu.VMEM((2 * D,), jnp.float32),
        pltpu.SemaphoreType.DMA,
        pltpu.SemaphoreType.DMA,
        pltpu.SemaphoreType.DMA,
    ],
)
def _sc_sums(node_hbm, region_hbm, idx_hbm, out_hbm,
             idx_v, buf0, buf1, buf2, row_v, out_v, sem0, sem1, sem2):
    wid = lax.axis_index("s") * NC + lax.axis_index("c")
    zeros = tuple(jnp.zeros((16,), jnp.float32) for _ in range(NV))

    rbase = wid * REG_W
    reg_pref = pltpu.async_copy(
        region_hbm.at[pl.ds(rbase, REG_CHUNK)], buf2.at[pl.ds(0, REG_CHUNK)],
        sem2)

    ibase = wid * B_W

    @pl.when(wid != NW - 1)
    def _():
        pltpu.sync_copy(idx_hbm.at[pl.ds(ibase, B_W)], idx_v)

    @pl.when(wid == NW - 1)
    def _():
        pltpu.sync_copy(idx_hbm.at[pl.ds(ibase, B_LAST)],
                        idx_v.at[pl.ds(0, B_LAST)])
        zi = jnp.zeros((16,), jnp.int32)
        for t in range(PAD // 16):
            idx_v[pl.ds(B_LAST + t * 16, 16)] = zi

    def gather(c, buf, sem):
        return pltpu.async_copy(
            node_hbm.at[idx_v.at[pl.ds(c * CHUNK, CHUNK)]], buf, sem)

    gather(0, buf0, sem0)
    gather(1, buf1, sem1)

    def bbody(p, a):
        c = p * 2
        for buf, sem, off in ((buf0, sem0, 0), (buf1, sem1, 1)):
            pltpu.make_async_copy(
                node_hbm.at[idx_v.at[pl.ds(0, CHUNK)]], buf, sem).wait()
            a = _accum(buf, CHUNK, a)

            @pl.when(c + off + 2 < N_CHUNKS)
            def _():
                gather(c + off + 2, buf, sem)
        return a

    acc_b = lax.fori_loop(0, N_CHUNKS // 2, bbody, zeros)

    pltpu.sync_copy(node_hbm.at[pl.ds(0, 1)], row_v)
    w = jnp.float32(PAD / NW)
    acc_b = tuple(acc_b[i] - w * row_v[0, pl.ds(i * 16, 16)]
                  for i in range(NV))

    rbufs = (buf2, buf0, buf1)
    rsems = (sem2, sem0, sem1)
    rhandles = [reg_pref, None, None]
    acc_r = zeros
    for c in range(REG_NCHUNKS):
        if c + 1 < REG_NCHUNKS:
            rhandles[c + 1] = pltpu.async_copy(
                region_hbm.at[pl.ds(rbase + (c + 1) * REG_CHUNK, REG_CHUNK)],
                rbufs[c + 1].at[pl.ds(0, REG_CHUNK)], rsems[c + 1])
        rhandles[c].wait()
        acc_r = _accum(rbufs[c], REG_CHUNK, acc_r)

    pltpu.sync_copy(
        region_hbm.at[pl.ds(REG_W * NW + lax.rem(wid, REG_TAIL), 1)], row_v)
    half = jnp.float32(0.5)
    acc_r = tuple(acc_r[i] + half * row_v[0, pl.ds(i * 16, 16)]
                  for i in range(NV))

    for i in range(NV):
        out_v[pl.ds(i * 16, 16)] = acc_r[i]
        out_v[pl.ds(D + i * 16, 16)] = acc_b[i]
    pltpu.sync_copy(out_v, out_hbm.at[wid])


def _head_body(partials, W1, b1, W2, b2, W3, b3, W4, b4, W5, b5, W6, b6, out):
    s = jnp.sum(partials[...], axis=0)
    gs = (s[:D] * jnp.float32(1.0 / N_REG)).reshape(1, D)
    bm = (s[D:] * jnp.float32(1.0 / B)).reshape(1, D)
    hi = lax.Precision.HIGHEST
    f32 = jnp.float32
    gs = jnp.maximum(jnp.dot(gs, W1[...], precision=hi, preferred_element_type=f32) + b1[...], 0.0)
    gs = jnp.maximum(jnp.dot(gs, W2[...], precision=hi, preferred_element_type=f32) + b2[...], 0.0)
    bi = jnp.maximum(jnp.dot(bm, W3[...], precision=hi, preferred_element_type=f32) + b3[...], 0.0)
    bi = jnp.dot(bi, W4[...], precision=hi, preferred_element_type=f32) + b4[...]
    w5 = W5[...]
    h = jnp.dot(gs, w5[:128], precision=hi, preferred_element_type=f32)
    h = h + jnp.dot(bi, w5[128:], precision=hi, preferred_element_type=f32)
    h = jnp.maximum(h + b5[...], 0.0)
    v = jnp.dot(h, W6[...], precision=hi, preferred_element_type=f32) + b6[...]
    out[...] = v


def kernel(node_embeddings, region_embeddings, boundary_nodes,
           W1, b1, W2, b2, W3, b3, W4, b4, W5, b5, W6, b6):
    idx = boundary_nodes.astype(jnp.int32)

    partials = _sc_sums(node_embeddings, region_embeddings, idx)

    out = pl.pallas_call(
        _head_body,
        out_shape=jax.ShapeDtypeStruct((1, 1), jnp.float32),
    )(partials, W1, b1, W2, b2, W3, b3, W4, b4, W5, b5, W6, b6)
    return out.reshape(1)


# speedup vs baseline: 1.1420x; 1.0357x over previous
import functools

import jax
import jax.numpy as jnp
from jax import lax
from jax.experimental import pallas as pl
from jax.experimental.pallas import tpu as pltpu
from jax.experimental.pallas import tpu_sc as plsc

D = 256
N_NODES = 100000
N_REG = 10000
B = 50000

NC = 2
NS = 16
NW = NC * NS

CHUNK = 112
N_CHUNKS = 14
B_W = CHUNK * N_CHUNKS
B_LAST = B - (NW - 1) * B_W
PAD = NW * B_W - B

REG_W = 312
REG_CHUNK = 104
REG_NCHUNKS = 3
REG_TAIL = N_REG - REG_W * NW

NV = D // 16


def _accum(buf_ref, nrows, acc):
    def body(r, a):
        return tuple(a[i] + buf_ref[r, pl.ds(i * 16, 16)] for i in range(NV))
    return lax.fori_loop(0, nrows, body, acc)


@functools.partial(
    pl.kernel,
    mesh=plsc.VectorSubcoreMesh(core_axis_name="c", subcore_axis_name="s"),
    out_type=jax.ShapeDtypeStruct((NW, 2 * D), jnp.float32),
    scratch_types=[
        pltpu.VMEM((B_W,), jnp.int32),
        pltpu.VMEM((CHUNK, D), jnp.float32),
        pltpu.VMEM((CHUNK, D), jnp.float32),
        pltpu.VMEM((CHUNK, D), jnp.float32),
        pltpu.VMEM((CHUNK, D), jnp.float32),
        pltpu.VMEM((1, D), jnp.float32),
        pltpu.VMEM((2 * D,), jnp.float32),
        pltpu.SemaphoreType.DMA,
        pltpu.SemaphoreType.DMA,
        pltpu.SemaphoreType.DMA,
        pltpu.SemaphoreType.DMA,
    ],
)
def _sc_sums(node_hbm, region_hbm, idx_hbm, out_hbm,
             idx_v, buf0, buf1, buf2, buf3, row_v, out_v,
             sem0, sem1, sem2, sem3):
    wid = lax.axis_index("s") * NC + lax.axis_index("c")
    zeros = tuple(jnp.zeros((16,), jnp.float32) for _ in range(NV))
    rbase = wid * REG_W

    ibase = wid * B_W

    @pl.when(wid != NW - 1)
    def _():
        pltpu.sync_copy(idx_hbm.at[pl.ds(ibase, B_W)], idx_v)

    @pl.when(wid == NW - 1)
    def _():
        pltpu.sync_copy(idx_hbm.at[pl.ds(ibase, B_LAST)],
                        idx_v.at[pl.ds(0, B_LAST)])
        zi = jnp.zeros((16,), jnp.int32)
        for t in range(PAD // 16):
            idx_v[pl.ds(B_LAST + t * 16, 16)] = zi

    bbufs = (buf0, buf1, buf2, buf3)
    bsems = (sem0, sem1, sem2, sem3)

    def gather(c, buf, sem):
        return pltpu.async_copy(
            node_hbm.at[idx_v.at[pl.ds(c * CHUNK, CHUNK)]], buf, sem)

    def gwait(buf, sem):
        pltpu.make_async_copy(
            node_hbm.at[idx_v.at[pl.ds(0, CHUNK)]], buf, sem).wait()

    for j in range(3):
        gather(j, bbufs[j], bsems[j])

    def bbody(p, a):
        c0 = p * 4
        for j in range(4):
            c = c0 + j
            jn = (j + 3) % 4
            gwait(bbufs[j], bsems[j])

            @pl.when(c + 3 < N_CHUNKS)
            def _():
                gather(c + 3, bbufs[jn], bsems[jn])

            a = _accum(bbufs[j], CHUNK, a)
        return a

    acc_b = lax.fori_loop(0, (N_CHUNKS - 2) // 4, bbody, zeros)

    rbufs = (buf2, buf3, buf0)
    rsems = (sem2, sem3, sem0)

    def rissue(c):
        pltpu.async_copy(
            region_hbm.at[pl.ds(rbase + c * REG_CHUNK, REG_CHUNK)],
            rbufs[c].at[pl.ds(0, REG_CHUNK)], rsems[c])

    def rwait(c):
        pltpu.make_async_copy(
            region_hbm.at[pl.ds(rbase, REG_CHUNK)],
            rbufs[c].at[pl.ds(0, REG_CHUNK)], rsems[c]).wait()

    rissue(0)
    rissue(1)

    for c in (N_CHUNKS - 2, N_CHUNKS - 1):
        j = c % 4
        gwait(bbufs[j], bsems[j])
        acc_b = _accum(bbufs[j], CHUNK, acc_b)

    pltpu.sync_copy(node_hbm.at[pl.ds(0, 1)], row_v)
    w = jnp.float32(PAD / NW)
    acc_b = tuple(acc_b[i] - w * row_v[0, pl.ds(i * 16, 16)]
                  for i in range(NV))

    acc_r = zeros
    for c in range(REG_NCHUNKS):
        if c + 2 < REG_NCHUNKS:
            rissue(c + 2)
        rwait(c)
        acc_r = _accum(rbufs[c], REG_CHUNK, acc_r)

    pltpu.sync_copy(
        region_hbm.at[pl.ds(REG_W * NW + lax.rem(wid, REG_TAIL), 1)], row_v)
    half = jnp.float32(0.5)
    acc_r = tuple(acc_r[i] + half * row_v[0, pl.ds(i * 16, 16)]
                  for i in range(NV))

    for i in range(NV):
        out_v[pl.ds(i * 16, 16)] = acc_r[i]
        out_v[pl.ds(D + i * 16, 16)] = acc_b[i]
    pltpu.sync_copy(out_v, out_hbm.at[wid])


def _head_body(partials, W1, b1, W2, b2, W3, b3, W4, b4, W5, b5, W6, b6, out):
    s = jnp.sum(partials[...], axis=0)
    gs = (s[:D] * jnp.float32(1.0 / N_REG)).reshape(1, D)
    bm = (s[D:] * jnp.float32(1.0 / B)).reshape(1, D)
    f32 = jnp.float32
    gs = jnp.maximum(jnp.dot(gs, W1[...], preferred_element_type=f32) + b1[...], 0.0)
    gs = jnp.maximum(jnp.dot(gs, W2[...], preferred_element_type=f32) + b2[...], 0.0)
    bi = jnp.maximum(jnp.dot(bm, W3[...], preferred_element_type=f32) + b3[...], 0.0)
    bi = jnp.dot(bi, W4[...], preferred_element_type=f32) + b4[...]
    w5 = W5[...]
    h = jnp.dot(gs, w5[:128], preferred_element_type=f32)
    h = h + jnp.dot(bi, w5[128:], preferred_element_type=f32)
    h = jnp.maximum(h + b5[...], 0.0)
    v = jnp.dot(h, W6[...], preferred_element_type=f32) + b6[...]
    out[...] = v


def kernel(node_embeddings, region_embeddings, boundary_nodes,
           W1, b1, W2, b2, W3, b3, W4, b4, W5, b5, W6, b6):
    idx = boundary_nodes.astype(jnp.int32)

    partials = _sc_sums(node_embeddings, region_embeddings, idx)

    out = pl.pallas_call(
        _head_body,
        out_shape=jax.ShapeDtypeStruct((1, 1), jnp.float32),
    )(partials, W1, b1, W2, b2, W3, b3, W4, b4, W5, b5, W6, b6)
    return out.reshape(1)
